# baseline (speedup 1.0000x reference)
import functools

import jax
import jax.numpy as jnp
from jax.experimental import pallas as pl
from jax.experimental.pallas import tpu as pltpu

_EPS = 1e-5
_ROW_CHUNK = 2048


def _pick_cz(n):
    for t in (2048, 1024, 512, 256, 128, 64, 32, 16, 8):
        if n % t == 0:
            return t
    return n


def _acc_stats(st_ref, z):
    ones = jnp.ones((1, z.shape[0]), jnp.float32)
    st_ref[0:1, :] += jnp.dot(ones, z, preferred_element_type=jnp.float32)
    st_ref[1:2, :] += jnp.dot(ones, z * z, preferred_element_type=jnp.float32)


def _fold(st_ref, g_ref, be_ref, ss_ref, inv_n):
    mu = st_ref[0:1, :] * inv_n
    var = st_ref[1:2, :] * inv_n - mu * mu
    scale = g_ref[...] * jax.lax.rsqrt(var + _EPS)
    ss_ref[0:1, :] = scale
    ss_ref[1:2, :] = be_ref[...] - mu * scale


def _mlp_bn_body(cz, nb, inv_n,
                 x_ref, w0_ref, w1_ref, w2_ref, w3_ref,
                 g0_ref, be0_ref, g1_ref, be1_ref,
                 g2_ref, be2_ref, g3_ref, be3_ref,
                 o_ref,
                 z2c, w0f, xbuf, obuf, st1, st2, st3, st4,
                 ss1, ss2, ss3, ss4, sx, so):
    p = pl.program_id(0)
    f3 = w2_ref.shape[1]

    def xdma(k, slot):
        return pltpu.make_async_copy(
            x_ref.at[pl.ds(k * cz, cz), :], xbuf.at[slot], sx.at[slot])

    def odma(k, slot):
        return pltpu.make_async_copy(
            obuf.at[slot], o_ref.at[pl.ds(k * cz, cz), :], so.at[slot])

    def x_loop(chunk_fn):
        xdma(0, 0).start()

        def it(k, c):
            slot = jax.lax.rem(k, 2)

            @pl.when(k + 1 < nb)
            def _():
                xdma(k + 1, jax.lax.rem(k + 1, 2)).start()

            xdma(k, slot).wait()
            xb = xbuf[slot].astype(jnp.bfloat16)
            xa = jnp.concatenate(
                [xb, jnp.ones((cz, 1), jnp.bfloat16)], axis=1)
            chunk_fn(k, xa)
            return c

        jax.lax.fori_loop(0, nb, it, 0)

    @pl.when(p == 0)
    def _():
        st1[...] = jnp.zeros_like(st1)
        w0f[0:w0_ref.shape[0], :] = w0_ref[...]
        w0f[w0_ref.shape[0]:w0_ref.shape[0] + 1, :] = jnp.zeros(
            (1, w0_ref.shape[1]), jnp.bfloat16)

        def chunk(k, xa):
            z1 = jnp.dot(xa, w0f[...], preferred_element_type=jnp.float32)
            _acc_stats(st1, z1)

        x_loop(chunk)
        _fold(st1, g0_ref, be0_ref, ss1, inv_n)
        w0f[0:w0_ref.shape[0], :] = (
            w0_ref[...].astype(jnp.float32) * ss1[0:1, :]).astype(jnp.bfloat16)
        w0f[w0_ref.shape[0]:w0_ref.shape[0] + 1, :] = (
            ss1[1:2, :].astype(jnp.bfloat16))

    @pl.when(p == 1)
    def _():
        st2[...] = jnp.zeros_like(st2)

        def chunk(k, xa):
            z1bn = jnp.dot(xa, w0f[...], preferred_element_type=jnp.float32)
            h1 = jnp.maximum(z1bn, 0.0).astype(jnp.bfloat16)
            z2 = jnp.dot(h1, w1_ref[...], preferred_element_type=jnp.float32)
            _acc_stats(st2, z2)
            z2c[pl.ds(k * cz, cz), :] = z2.astype(jnp.bfloat16)

        x_loop(chunk)
        _fold(st2, g1_ref, be1_ref, ss2, inv_n)

    @pl.when(p == 2)
    def _():
        st3[...] = jnp.zeros_like(st3)

        def it(k, c):
            z2 = z2c[pl.ds(k * cz, cz), :].astype(jnp.float32)
            h2 = jnp.maximum(z2 * ss2[0:1, :] + ss2[1:2, :], 0.0)
            h2 = h2.astype(jnp.bfloat16)
            z3 = jnp.dot(h2, w2_ref[...], preferred_element_type=jnp.float32)
            _acc_stats(st3, z3)
            z2c[pl.ds(k * cz, cz), 0:f3] = z3.astype(jnp.bfloat16)
            return c

        jax.lax.fori_loop(0, nb, it, 0)
        _fold(st3, g2_ref, be2_ref, ss3, inv_n)

    @pl.when(p == 3)
    def _():
        st4[...] = jnp.zeros_like(st4)

        def it(k, c):
            z3 = z2c[pl.ds(k * cz, cz), 0:f3].astype(jnp.float32)
            h3 = jnp.maximum(z3 * ss3[0:1, :] + ss3[1:2, :],
                             0.0).astype(jnp.bfloat16)
            z2c[pl.ds(k * cz, cz), 0:f3] = h3
            z4 = jnp.dot(h3, w3_ref[...], preferred_element_type=jnp.float32)
            _acc_stats(st4, z4)
            return c

        jax.lax.fori_loop(0, nb, it, 0)
        _fold(st4, g3_ref, be3_ref, ss4, inv_n)

    @pl.when(p == 4)
    def _():
        def it(k, c):
            slot = jax.lax.rem(k, 2)

            @pl.when(k >= 2)
            def _():
                odma(k - 2, slot).wait()

            h3 = z2c[pl.ds(k * cz, cz), 0:f3]
            z4 = jnp.dot(h3, w3_ref[...], preferred_element_type=jnp.float32)
            obuf[slot] = z4 * ss4[0:1, :] + ss4[1:2, :]
            odma(k, slot).start()
            return c

        jax.lax.fori_loop(0, nb, it, 0)
        if nb >= 2:
            odma(nb - 2, (nb - 2) % 2).wait()
        odma(nb - 1, (nb - 1) % 2).wait()


def kernel(x, w0, b0, g0, be0, w1, b1, g1, be1, w2, b2, g2, be2,
           w3, b3, g3, be3):
    n, f_in = x.shape
    f1, f2, f3, f4 = w0.shape[1], w1.shape[1], w2.shape[1], w3.shape[1]
    cz = _pick_cz(n)
    nb = n // cz

    w0b, w1b, w2b, w3b = (w.astype(jnp.bfloat16) for w in (w0, w1, w2, w3))

    fixed = lambda p: (0, 0)
    body = functools.partial(_mlp_bn_body, cz, nb, 1.0 / n)

    return pl.pallas_call(
        body,
        out_shape=jax.ShapeDtypeStruct((n, f4), jnp.float32),
        grid=(5,),
        in_specs=[
            pl.BlockSpec(memory_space=pl.ANY),
            pl.BlockSpec((f_in, f1), fixed),
            pl.BlockSpec((f1, f2), fixed),
            pl.BlockSpec((f2, f3), fixed),
            pl.BlockSpec((f3, f4), fixed),
            pl.BlockSpec((1, f1), fixed), pl.BlockSpec((1, f1), fixed),
            pl.BlockSpec((1, f2), fixed), pl.BlockSpec((1, f2), fixed),
            pl.BlockSpec((1, f3), fixed), pl.BlockSpec((1, f3), fixed),
            pl.BlockSpec((1, f4), fixed), pl.BlockSpec((1, f4), fixed),
        ],
        out_specs=pl.BlockSpec(memory_space=pl.ANY),
        scratch_shapes=[
            pltpu.VMEM((n, f2), jnp.bfloat16),
            pltpu.VMEM((f_in + 1, f1), jnp.bfloat16),
            pltpu.VMEM((2, cz, f_in), jnp.float32),
            pltpu.VMEM((2, cz, f4), jnp.float32),
            pltpu.VMEM((2, f1), jnp.float32),
            pltpu.VMEM((2, f2), jnp.float32),
            pltpu.VMEM((2, f3), jnp.float32),
            pltpu.VMEM((2, f4), jnp.float32),
            pltpu.VMEM((2, f1), jnp.float32),
            pltpu.VMEM((2, f2), jnp.float32),
            pltpu.VMEM((2, f3), jnp.float32),
            pltpu.VMEM((2, f4), jnp.float32),
            pltpu.SemaphoreType.DMA((2,)),
            pltpu.SemaphoreType.DMA((2,)),
        ],
        compiler_params=pltpu.CompilerParams(
            dimension_semantics=("arbitrary",),
            vmem_limit_bytes=60000 * 1024,
        ),
        name="mlp_bn4_fused",
    )(x, w0b, w1b, w2b, w3b, g0, be0, g1, be1, g2, be2, g3, be3)


# speedup vs baseline: 1.0382x; 1.0382x over previous
import functools

import jax
import jax.numpy as jnp
from jax.experimental import pallas as pl
from jax.experimental.pallas import tpu as pltpu

_EPS = 1e-5
_ROW_CHUNK = 2048


def _pick_tm(n):
    for t in (8192, 4096, 2048, 1024, 512, 256, 128, 64, 32, 16, 8):
        if n % t == 0:
            return t
    return n


def _acc_stats(st_ref, z):
    ones = jnp.ones((1, z.shape[0]), jnp.float32)
    st_ref[0:1, :] += jnp.dot(ones, z, preferred_element_type=jnp.float32)
    st_ref[1:2, :] += jnp.dot(ones, z * z, preferred_element_type=jnp.float32)


def _fold(st_ref, g_ref, be_ref, ss_ref, inv_n):
    mu = st_ref[0:1, :] * inv_n
    var = st_ref[1:2, :] * inv_n - mu * mu
    scale = g_ref[...] * jax.lax.rsqrt(var + _EPS)
    ss_ref[0:1, :] = scale
    ss_ref[1:2, :] = be_ref[...] - mu * scale


def _mlp_bn_body(tm, nb, inv_n,
                 x_ref, w0_ref, w1_ref, w2_ref, w3_ref,
                 g0_ref, be0_ref, g1_ref, be1_ref,
                 g2_ref, be2_ref, g3_ref, be3_ref,
                 o_ref,
                 z2c, w0f, obuf, st1, st2, st3, st4, ss1, ss2, ss3, ss4, so):
    p = pl.program_id(0)
    i = pl.program_id(1)
    f3 = w2_ref.shape[1]
    cz = min(_ROW_CHUNK, tm)
    chunks = range(0, tm, cz)

    def xb_aug(r):
        xb = x_ref[pl.ds(r, cz), :].astype(jnp.bfloat16)
        return jnp.concatenate(
            [xb, jnp.ones((cz, 1), jnp.bfloat16)], axis=1)

    @pl.when(p == 0)
    def _():
        @pl.when(i == 0)
        def _():
            st1[...] = jnp.zeros_like(st1)
            w0f[0:w0_ref.shape[0], :] = w0_ref[...]
            w0f[w0_ref.shape[0]:w0_ref.shape[0] + 1, :] = jnp.zeros(
                (1, w0_ref.shape[1]), jnp.bfloat16)

        for r in chunks:
            z1 = jnp.dot(xb_aug(r), w0f[...],
                         preferred_element_type=jnp.float32)
            _acc_stats(st1, z1)

        @pl.when(i == nb - 1)
        def _():
            _fold(st1, g0_ref, be0_ref, ss1, inv_n)
            w0f[0:w0_ref.shape[0], :] = (
                w0_ref[...].astype(jnp.float32)
                * ss1[0:1, :]).astype(jnp.bfloat16)
            w0f[w0_ref.shape[0]:w0_ref.shape[0] + 1, :] = (
                ss1[1:2, :].astype(jnp.bfloat16))

    @pl.when(p == 1)
    def _():
        @pl.when(i == 0)
        def _():
            st2[...] = jnp.zeros_like(st2)

        for r in chunks:
            z1bn = jnp.dot(xb_aug(r), w0f[...],
                           preferred_element_type=jnp.float32)
            h1 = jnp.maximum(z1bn, 0.0).astype(jnp.bfloat16)
            z2 = jnp.dot(h1, w1_ref[...], preferred_element_type=jnp.float32)
            _acc_stats(st2, z2)
            z2c[pl.ds(i * tm + r, cz), :] = z2.astype(jnp.bfloat16)

        @pl.when(i == nb - 1)
        def _():
            _fold(st2, g1_ref, be1_ref, ss2, inv_n)

    @pl.when(p == 2)
    def _():
        @pl.when(i == 0)
        def _():
            st3[...] = jnp.zeros_like(st3)

        for r in chunks:
            z2 = z2c[pl.ds(i * tm + r, cz), :].astype(jnp.float32)
            h2 = jnp.maximum(z2 * ss2[0:1, :] + ss2[1:2, :], 0.0)
            h2 = h2.astype(jnp.bfloat16)
            z3 = jnp.dot(h2, w2_ref[...], preferred_element_type=jnp.float32)
            _acc_stats(st3, z3)
            z2c[pl.ds(i * tm + r, cz), 0:f3] = z3.astype(jnp.bfloat16)

        @pl.when(i == nb - 1)
        def _():
            _fold(st3, g2_ref, be2_ref, ss3, inv_n)

    @pl.when(p == 3)
    def _():
        @pl.when(i == 0)
        def _():
            st4[...] = jnp.zeros_like(st4)

        for r in chunks:
            z3 = z2c[pl.ds(i * tm + r, cz), 0:f3].astype(jnp.float32)
            h3 = jnp.maximum(z3 * ss3[0:1, :] + ss3[1:2, :],
                             0.0).astype(jnp.bfloat16)
            z2c[pl.ds(i * tm + r, cz), 0:f3] = h3
            z4 = jnp.dot(h3, w3_ref[...], preferred_element_type=jnp.float32)
            _acc_stats(st4, z4)

        @pl.when(i == nb - 1)
        def _():
            _fold(st4, g3_ref, be3_ref, ss4, inv_n)

    @pl.when(p == 4)
    def _():
        def odma(r, slot):
            return pltpu.make_async_copy(
                obuf.at[slot], o_ref.at[pl.ds(i * tm + r, cz), :],
                so.at[slot])

        nch = len(chunks)
        for c, r in enumerate(chunks):
            slot = c % 2
            if c >= 2:
                odma(r - 2 * cz, slot).wait()
            elif nch >= 2:
                @pl.when(i > 0)
                def _():
                    odma(r, slot).wait()

            h3 = z2c[pl.ds(i * tm + r, cz), 0:f3]
            z4 = jnp.dot(h3, w3_ref[...], preferred_element_type=jnp.float32)
            obuf[slot] = z4 * ss4[0:1, :] + ss4[1:2, :]
            odma(r, slot).start()
            if nch < 2:
                odma(r, slot).wait()

        if nch >= 2:
            @pl.when(i == nb - 1)
            def _():
                odma(chunks[-2], (nch - 2) % 2).wait()
                odma(chunks[-1], (nch - 1) % 2).wait()


def kernel(x, w0, b0, g0, be0, w1, b1, g1, be1, w2, b2, g2, be2,
           w3, b3, g3, be3):
    n, f_in = x.shape
    f1, f2, f3, f4 = w0.shape[1], w1.shape[1], w2.shape[1], w3.shape[1]
    tm = _pick_tm(n)
    nb = n // tm

    w0b, w1b, w2b, w3b = (w.astype(jnp.bfloat16) for w in (w0, w1, w2, w3))

    fixed = lambda p, i: (0, 0)
    body = functools.partial(_mlp_bn_body, tm, nb, 1.0 / n)

    return pl.pallas_call(
        body,
        out_shape=jax.ShapeDtypeStruct((n, f4), jnp.float32),
        grid=(5, nb),
        in_specs=[
            pl.BlockSpec((tm, f_in), lambda p, i: (jnp.where(p < 2, i, 0), 0)),
            pl.BlockSpec((f_in, f1), fixed),
            pl.BlockSpec((f1, f2), fixed),
            pl.BlockSpec((f2, f3), fixed),
            pl.BlockSpec((f3, f4), fixed),
            pl.BlockSpec((1, f1), fixed), pl.BlockSpec((1, f1), fixed),
            pl.BlockSpec((1, f2), fixed), pl.BlockSpec((1, f2), fixed),
            pl.BlockSpec((1, f3), fixed), pl.BlockSpec((1, f3), fixed),
            pl.BlockSpec((1, f4), fixed), pl.BlockSpec((1, f4), fixed),
        ],
        out_specs=pl.BlockSpec(memory_space=pl.ANY),
        scratch_shapes=[
            pltpu.VMEM((n, f2), jnp.bfloat16),
            pltpu.VMEM((f_in + 1, f1), jnp.bfloat16),
            pltpu.VMEM((2, min(_ROW_CHUNK, tm), f4), jnp.float32),
            pltpu.VMEM((2, f1), jnp.float32),
            pltpu.VMEM((2, f2), jnp.float32),
            pltpu.VMEM((2, f3), jnp.float32),
            pltpu.VMEM((2, f4), jnp.float32),
            pltpu.VMEM((2, f1), jnp.float32),
            pltpu.VMEM((2, f2), jnp.float32),
            pltpu.VMEM((2, f3), jnp.float32),
            pltpu.VMEM((2, f4), jnp.float32),
            pltpu.SemaphoreType.DMA((2,)),
        ],
        compiler_params=pltpu.CompilerParams(
            dimension_semantics=("arbitrary", "arbitrary"),
            vmem_limit_bytes=60000 * 1024,
        ),
        name="mlp_bn4_fused",
    )(x, w0b, w1b, w2b, w3b, g0, be0, g1, be1, g2, be2, g3, be3)


# speedup vs baseline: 1.0873x; 1.0473x over previous
import functools

import jax
import jax.numpy as jnp
from jax.experimental import pallas as pl
from jax.experimental.pallas import tpu as pltpu

_EPS = 1e-5
_ROW_CHUNK = 2048


def _pick_tm(n):
    for t in (4096, 2048, 1024, 512, 256, 128, 64, 32, 16, 8):
        if n % t == 0:
            return t
    return n


def _acc_stats(st_ref, z):
    ones = jnp.ones((1, z.shape[0]), jnp.float32)
    st_ref[0:1, :] += jnp.dot(ones, z, preferred_element_type=jnp.float32)
    st_ref[1:2, :] += jnp.dot(ones, z * z, preferred_element_type=jnp.float32)


def _fold(st_ref, g_ref, be_ref, ss_ref, inv_n):
    mu = st_ref[0:1, :] * inv_n
    var = st_ref[1:2, :] * inv_n - mu * mu
    scale = g_ref[...] * jax.lax.rsqrt(var + _EPS)
    ss_ref[0:1, :] = scale
    ss_ref[1:2, :] = be_ref[...] - mu * scale


def _mlp_bn_body(tm, nb, inv_n,
                 x_ref, w0_ref, w1_ref, w2_ref, w3_ref,
                 g0_ref, be0_ref, g1_ref, be1_ref,
                 g2_ref, be2_ref, g3_ref, be3_ref,
                 o_ref,
                 z2c, w0f, st1, st2, st3, st4, ss1, ss2, ss3, ss4):
    p = pl.program_id(0)
    i = pl.program_id(1)
    f3 = w2_ref.shape[1]
    cz = min(_ROW_CHUNK, tm)
    chunks = range(0, tm, cz)

    def xb_aug(r):
        xb = x_ref[pl.ds(r, cz), :].astype(jnp.bfloat16)
        return jnp.concatenate(
            [xb, jnp.ones((cz, 1), jnp.bfloat16)], axis=1)

    @pl.when(p == 0)
    def _():
        @pl.when(i == 0)
        def _():
            st1[...] = jnp.zeros_like(st1)
            w0f[0:w0_ref.shape[0], :] = w0_ref[...]
            w0f[w0_ref.shape[0]:w0_ref.shape[0] + 1, :] = jnp.zeros(
                (1, w0_ref.shape[1]), jnp.bfloat16)

        for r in chunks:
            z1 = jnp.dot(xb_aug(r), w0f[...],
                         preferred_element_type=jnp.float32)
            _acc_stats(st1, z1)

        @pl.when(i == nb - 1)
        def _():
            _fold(st1, g0_ref, be0_ref, ss1, inv_n)
            w0f[0:w0_ref.shape[0], :] = (
                w0_ref[...].astype(jnp.float32)
                * ss1[0:1, :]).astype(jnp.bfloat16)
            w0f[w0_ref.shape[0]:w0_ref.shape[0] + 1, :] = (
                ss1[1:2, :].astype(jnp.bfloat16))

    @pl.when(p == 1)
    def _():
        @pl.when(i == 0)
        def _():
            st2[...] = jnp.zeros_like(st2)

        for r in chunks:
            z1bn = jnp.dot(xb_aug(r), w0f[...],
                           preferred_element_type=jnp.float32)
            h1 = jnp.maximum(z1bn, 0.0).astype(jnp.bfloat16)
            z2 = jnp.dot(h1, w1_ref[...], preferred_element_type=jnp.float32)
            _acc_stats(st2, z2)
            z2c[pl.ds(i * tm + r, cz), :] = z2.astype(jnp.bfloat16)

        @pl.when(i == nb - 1)
        def _():
            _fold(st2, g1_ref, be1_ref, ss2, inv_n)

    @pl.when(p == 2)
    def _():
        @pl.when(i == 0)
        def _():
            st3[...] = jnp.zeros_like(st3)

        for r in chunks:
            z2 = z2c[pl.ds(i * tm + r, cz), :].astype(jnp.float32)
            h2 = jnp.maximum(z2 * ss2[0:1, :] + ss2[1:2, :], 0.0)
            h2 = h2.astype(jnp.bfloat16)
            z3 = jnp.dot(h2, w2_ref[...], preferred_element_type=jnp.float32)
            _acc_stats(st3, z3)
            z2c[pl.ds(i * tm + r, cz), 0:f3] = z3.astype(jnp.bfloat16)

        @pl.when(i == nb - 1)
        def _():
            _fold(st3, g2_ref, be2_ref, ss3, inv_n)

    @pl.when(p == 3)
    def _():
        @pl.when(i == 0)
        def _():
            st4[...] = jnp.zeros_like(st4)

        for r in chunks:
            z3 = z2c[pl.ds(i * tm + r, cz), 0:f3].astype(jnp.float32)
            h3 = jnp.maximum(z3 * ss3[0:1, :] + ss3[1:2, :],
                             0.0).astype(jnp.bfloat16)
            z2c[pl.ds(i * tm + r, cz), 0:f3] = h3
            z4 = jnp.dot(h3, w3_ref[...], preferred_element_type=jnp.float32)
            _acc_stats(st4, z4)

        @pl.when(i == nb - 1)
        def _():
            _fold(st4, g3_ref, be3_ref, ss4, inv_n)

    @pl.when(p == 4)
    def _():
        for r in chunks:
            h3 = z2c[pl.ds(i * tm + r, cz), 0:f3]
            z4 = jnp.dot(h3, w3_ref[...], preferred_element_type=jnp.float32)
            o_ref[pl.ds(r, cz), :] = z4 * ss4[0:1, :] + ss4[1:2, :]


def kernel(x, w0, b0, g0, be0, w1, b1, g1, be1, w2, b2, g2, be2,
           w3, b3, g3, be3):
    n, f_in = x.shape
    f1, f2, f3, f4 = w0.shape[1], w1.shape[1], w2.shape[1], w3.shape[1]
    tm = _pick_tm(n)
    nb = n // tm

    w0b, w1b, w2b, w3b = (w.astype(jnp.bfloat16) for w in (w0, w1, w2, w3))

    fixed = lambda p, i: (0, 0)
    body = functools.partial(_mlp_bn_body, tm, nb, 1.0 / n)

    return pl.pallas_call(
        body,
        out_shape=jax.ShapeDtypeStruct((n, f4), jnp.float32),
        grid=(5, nb),
        in_specs=[
            pl.BlockSpec((tm, f_in), lambda p, i: (jnp.where(p < 2, i, 0), 0)),
            pl.BlockSpec((f_in, f1), fixed),
            pl.BlockSpec((f1, f2), fixed),
            pl.BlockSpec((f2, f3), fixed),
            pl.BlockSpec((f3, f4), fixed),
            pl.BlockSpec((1, f1), fixed), pl.BlockSpec((1, f1), fixed),
            pl.BlockSpec((1, f2), fixed), pl.BlockSpec((1, f2), fixed),
            pl.BlockSpec((1, f3), fixed), pl.BlockSpec((1, f3), fixed),
            pl.BlockSpec((1, f4), fixed), pl.BlockSpec((1, f4), fixed),
        ],
        out_specs=pl.BlockSpec((tm, f4),
                               lambda p, i: (jnp.where(p == 4, i, 0), 0)),
        scratch_shapes=[
            pltpu.VMEM((n, f2), jnp.bfloat16),
            pltpu.VMEM((f_in + 1, f1), jnp.bfloat16),
            pltpu.VMEM((2, f1), jnp.float32),
            pltpu.VMEM((2, f2), jnp.float32),
            pltpu.VMEM((2, f3), jnp.float32),
            pltpu.VMEM((2, f4), jnp.float32),
            pltpu.VMEM((2, f1), jnp.float32),
            pltpu.VMEM((2, f2), jnp.float32),
            pltpu.VMEM((2, f3), jnp.float32),
            pltpu.VMEM((2, f4), jnp.float32),
        ],
        compiler_params=pltpu.CompilerParams(
            dimension_semantics=("arbitrary", "arbitrary"),
            vmem_limit_bytes=60000 * 1024,
        ),
        name="mlp_bn4_fused",
    )(x, w0b, w1b, w2b, w3b, g0, be0, g1, be1, g2, be2, g3, be3)
